# Initial kernel scaffold; baseline (speedup 1.0000x reference)
import jax
import jax.numpy as jnp
from jax.experimental import pallas as pl


def kernel(inputs, table):
    raise NotImplementedError("write your pallas kernel here")


# speedup vs baseline: 4.9437x; 4.9437x over previous
import functools

import jax
import jax.numpy as jnp
from jax import lax
from jax.experimental import pallas as pl
from jax.experimental.pallas import tpu as pltpu
from jax.experimental.pallas import tpu_sc as plsc

EMB = 32
N = 16384 * 200
NC, NS = 2, 16
NW = NC * NS
ROW = 128
ROWS_TOTAL = N // ROW
ROWS_PER_W = ROWS_TOTAL // NW
K = 16
CHUNKS = ROWS_PER_W // K

_mesh = plsc.VectorSubcoreMesh(core_axis_name="c", subcore_axis_name="s")


@functools.partial(
    pl.kernel,
    mesh=_mesh,
    out_type=jax.ShapeDtypeStruct((ROWS_TOTAL, ROW, EMB), jnp.float32),
    scratch_types=[
        pltpu.VMEM((K, ROW), jnp.int32),
        pltpu.VMEM((K, ROW, EMB), jnp.float32),
        pltpu.SemaphoreType.DMA,
    ],
    compiler_params=pltpu.CompilerParams(use_tc_tiling_on_sc=False),
)
def _emb_lookup(idx_hbm, table_hbm, out_hbm, idx_v, rows_v, sem):
    wid = lax.axis_index("s") * NC + lax.axis_index("c")
    row0 = wid * ROWS_PER_W

    def chunk(g, carry):
        r = row0 + g * K
        pltpu.sync_copy(idx_hbm.at[pl.ds(r, K)], idx_v)
        copies = [
            pltpu.async_copy(table_hbm.at[idx_v.at[j]], rows_v.at[j], sem)
            for j in range(K)
        ]
        for c in copies:
            c.wait()
        pltpu.sync_copy(rows_v, out_hbm.at[pl.ds(r, K)])
        return carry

    lax.fori_loop(0, CHUNKS, chunk, 0)


def kernel(inputs, table):
    idx = inputs.reshape(ROWS_TOTAL, ROW)
    out = _emb_lookup(idx, table)
    return out.reshape(inputs.shape[0], inputs.shape[1], EMB)


# speedup vs baseline: 5.0341x; 1.0183x over previous
import functools

import jax
import jax.numpy as jnp
from jax import lax
from jax.experimental import pallas as pl
from jax.experimental.pallas import tpu as pltpu
from jax.experimental.pallas import tpu_sc as plsc

EMB = 32
N = 16384 * 200
NC, NS = 2, 16
NW = NC * NS
ROW = 128
ROWS_TOTAL = N // ROW
ROWS_PER_W = ROWS_TOTAL // NW
K = 8
NBUF = 2
CHUNKS = ROWS_PER_W // K
T = CHUNKS // NBUF

_mesh = plsc.VectorSubcoreMesh(core_axis_name="c", subcore_axis_name="s")


@functools.partial(
    pl.kernel,
    mesh=_mesh,
    out_type=jax.ShapeDtypeStruct((ROWS_TOTAL, ROW, EMB), jnp.float32),
    scratch_types=[
        pltpu.VMEM((NBUF, K, ROW), jnp.int32),
        pltpu.VMEM((NBUF, K, ROW, EMB), jnp.float32),
        pltpu.SemaphoreType.DMA,
        pltpu.SemaphoreType.DMA,
        pltpu.SemaphoreType.DMA,
        pltpu.SemaphoreType.DMA,
        pltpu.SemaphoreType.DMA,
    ],
    compiler_params=pltpu.CompilerParams(use_tc_tiling_on_sc=False),
)
def _emb_lookup(idx_hbm, table_hbm, out_hbm, idx_v, rows_v,
                isem0, isem1, gsem, osem0, osem1):
    wid = lax.axis_index("s") * NC + lax.axis_index("c")
    row0 = wid * ROWS_PER_W
    isem = (isem0, isem1)
    osem = (osem0, osem1)

    for b in range(NBUF):
        pltpu.async_copy(idx_hbm.at[pl.ds(row0 + b * K, K)], idx_v.at[b],
                         isem[b])

    def outer(t, carry):
        for b in range(NBUF):
            g = t * NBUF + b
            r = row0 + g * K
            pltpu.make_async_copy(idx_hbm.at[pl.ds(r, K)], idx_v.at[b],
                                  isem[b]).wait()

            @pl.when(t > 0)
            def _():
                pltpu.make_async_copy(rows_v.at[b], out_hbm.at[pl.ds(r, K)],
                                      osem[b]).wait()

            copies = [
                pltpu.async_copy(table_hbm.at[idx_v.at[b, j]],
                                 rows_v.at[b, j], gsem)
                for j in range(K)
            ]
            for c in copies:
                c.wait()

            @pl.when(t < T - 1)
            def _():
                pltpu.async_copy(idx_hbm.at[pl.ds(r + NBUF * K, K)],
                                 idx_v.at[b], isem[b])

            pltpu.async_copy(rows_v.at[b], out_hbm.at[pl.ds(r, K)], osem[b])
        return carry

    lax.fori_loop(0, T, outer, 0)

    for b in range(NBUF):
        r = row0 + ((T - 1) * NBUF + b) * K
        pltpu.make_async_copy(rows_v.at[b], out_hbm.at[pl.ds(r, K)],
                              osem[b]).wait()


def kernel(inputs, table):
    idx = inputs.reshape(ROWS_TOTAL, ROW)
    out = _emb_lookup(idx, table)
    return out.reshape(inputs.shape[0], inputs.shape[1], EMB)
